# baseline (speedup 1.0000x reference)
import functools

import jax
import jax.numpy as jnp
from jax import lax
from jax.experimental import pallas as pl
from jax.experimental.pallas import tpu as pltpu
from jax.experimental.pallas import tpu_sc as plsc

_B, _T, _D, _DH = 16, 4096, 1024, 64
_N = _B * _T
_NC, _NS = 2, 16
_NW = _NC * _NS
_RPW = _N // _NW
_CH = 128
_NCH = _RPW // _CH

_ROWS = 512


def _gather_body(tab_hbm, ids_hbm, out_hbm, idx_v, rows_v, gsem0, gsem1):
    wid = lax.axis_index("s") * _NC + lax.axis_index("c")
    base = wid * _RPW
    pltpu.sync_copy(ids_hbm.at[wid], idx_v)
    sems = (gsem0, gsem1)
    cps = [None, None]
    cps[0] = pltpu.async_copy(tab_hbm.at[idx_v.at[0]], rows_v.at[0], sems[0])
    for j in range(_NCH):
        if j + 1 < _NCH:
            b = (j + 1) % 2
            cps[b] = pltpu.async_copy(
                tab_hbm.at[idx_v.at[j + 1]], rows_v.at[b], sems[b])
        cps[j % 2].wait()
        pltpu.sync_copy(rows_v.at[j % 2],
                        out_hbm.at[pl.ds(base + j * _CH, _CH)])


def _token_gather(token_table, token_ids):
    ids3 = token_ids.reshape(_NW, _NCH, _CH)
    mesh = plsc.VectorSubcoreMesh(core_axis_name="c", subcore_axis_name="s")
    run = pl.kernel(
        _gather_body,
        out_type=jax.ShapeDtypeStruct((_N, _DH), jnp.bfloat16),
        mesh=mesh,
        scratch_types=[
            pltpu.VMEM((_NCH, _CH), jnp.int32),
            pltpu.VMEM((2, _CH, _DH), jnp.bfloat16),
            pltpu.SemaphoreType.DMA,
            pltpu.SemaphoreType.DMA,
        ],
        compiler_params=pltpu.CompilerParams(use_tc_tiling_on_sc=False),
    )
    return run(token_table.astype(jnp.bfloat16), ids3)


def _mlp_body(tok, aux, M74, W2, b2, W3, b3, g, beta, out):
    tk = tok[...]
    av = aux[...]
    f = av[:, 0:4].astype(jnp.bfloat16)
    idv = lax.bitcast_convert_type(av[:, 4:7], jnp.int32)
    order = (idv[:, 0:1] == idv[:, 1:2]).astype(jnp.bfloat16)
    iot = lax.broadcasted_iota(jnp.int32, (_ROWS, 4), 1)
    oh = (idv[:, 2:3] == iot).astype(jnp.bfloat16)
    ones = jnp.ones((_ROWS, 1), jnp.bfloat16)
    small = jnp.concatenate([f, order, oh, ones], axis=1)
    x = jnp.concatenate([tk, small], axis=1)
    p = jnp.dot(x, M74[...], preferred_element_type=jnp.float32)
    h = jnp.where(p >= 0, p, 0.01 * p).astype(jnp.bfloat16)
    h = jnp.dot(h, W2[...], preferred_element_type=jnp.float32) + b2[...]
    h = jnp.where(h >= 0, h, 0.01 * h).astype(jnp.bfloat16)
    h = jnp.dot(h, W3[...], preferred_element_type=jnp.float32) + b3[...]
    mu = jnp.mean(h, axis=1, keepdims=True)
    hc = h - mu
    var = jnp.mean(hc * hc, axis=1, keepdims=True)
    out[...] = hc * lax.rsqrt(var + 1e-15) * g[...] + beta[...]


def _mlp_call(tok, aux, M74, W2, b2, W3, b3, g, beta):
    row = lambda i: (i, 0)
    rep = lambda i: (0, 0)
    return pl.pallas_call(
        _mlp_body,
        grid=(_N // _ROWS,),
        in_specs=[
            pl.BlockSpec((_ROWS, _DH), row),
            pl.BlockSpec((_ROWS, 7), row),
            pl.BlockSpec((74, _D // 2), rep),
            pl.BlockSpec((_D // 2, _D), rep),
            pl.BlockSpec((1, _D), rep),
            pl.BlockSpec((_D, _D), rep),
            pl.BlockSpec((1, _D), rep),
            pl.BlockSpec((1, _D), rep),
            pl.BlockSpec((1, _D), rep),
        ],
        out_specs=pl.BlockSpec((_ROWS, _D), row),
        out_shape=jax.ShapeDtypeStruct((_N, _D), jnp.float32),
    )(tok, aux, M74, W2, b2, W3, b3, g, beta)


def kernel(feat, padded_index, etype_ids, token_ids, edge_W, edge_b,
           order_table, etype_table, token_table,
           W1, b1, W2, b2, W3, b3, ln_gamma, ln_beta):
    tok = _token_gather(token_table, token_ids)

    W1a, W1b, W1c, W1d = W1[:64], W1[64:128], W1[128:192], W1[192:]
    Wf = edge_W @ W1a
    dWo = ((order_table[1] - order_table[0]) @ W1b)[None, :]
    We = etype_table @ W1c
    bias1 = (b1 + edge_b @ W1a + order_table[0] @ W1b)[None, :]
    M74 = jnp.concatenate([W1d, Wf, dWo, We, bias1],
                          axis=0).astype(jnp.bfloat16)

    aux = jnp.concatenate(
        [feat.reshape(_N, 4),
         lax.bitcast_convert_type(padded_index.reshape(_N, 2), jnp.float32),
         lax.bitcast_convert_type(etype_ids.reshape(_N, 1), jnp.float32)],
        axis=1)

    out = _mlp_call(tok, aux, M74,
                    W2.astype(jnp.bfloat16), b2[None, :],
                    W3.astype(jnp.bfloat16), b3[None, :],
                    ln_gamma[None, :], ln_beta[None, :])
    return out.reshape(_B, _T, _D)


# speedup vs baseline: 1.1614x; 1.1614x over previous
import functools

import jax
import jax.numpy as jnp
from jax import lax
from jax.experimental import pallas as pl
from jax.experimental.pallas import tpu as pltpu
from jax.experimental.pallas import tpu_sc as plsc

_B, _T, _D, _DH = 16, 4096, 1024, 64
_N = _B * _T
_NC, _NS = 2, 16
_NW = _NC * _NS
_RPW = _N // _NW
_CH = 128
_NCH = _RPW // _CH

_ROWS = 512


def _gather_body(tab_hbm, ids_hbm, out_hbm, idx_v, rows_v, gsem0, gsem1):
    wid = lax.axis_index("s") * _NC + lax.axis_index("c")
    base = wid * _RPW
    pltpu.sync_copy(ids_hbm.at[wid], idx_v)
    sems = (gsem0, gsem1)
    cps = [None, None]
    cps[0] = pltpu.async_copy(tab_hbm.at[idx_v.at[0]], rows_v.at[0], sems[0])
    for j in range(_NCH):
        if j + 1 < _NCH:
            b = (j + 1) % 2
            cps[b] = pltpu.async_copy(
                tab_hbm.at[idx_v.at[j + 1]], rows_v.at[b], sems[b])
        cps[j % 2].wait()
        pltpu.sync_copy(rows_v.at[j % 2],
                        out_hbm.at[pl.ds(base + j * _CH, _CH)])


def _token_gather(token_table, token_ids):
    ids3 = token_ids.reshape(_NW, _NCH, _CH)
    mesh = plsc.VectorSubcoreMesh(core_axis_name="c", subcore_axis_name="s")
    run = pl.kernel(
        _gather_body,
        out_type=jax.ShapeDtypeStruct((_N, _DH), jnp.float32),
        mesh=mesh,
        scratch_types=[
            pltpu.VMEM((_NCH, _CH), jnp.int32),
            pltpu.VMEM((2, _CH, _DH), jnp.float32),
            pltpu.SemaphoreType.DMA,
            pltpu.SemaphoreType.DMA,
        ],
        compiler_params=pltpu.CompilerParams(use_tc_tiling_on_sc=False),
    )
    return run(token_table, ids3)


def _mlp_body(tok, aux, M74, W2, b2, W3, b3, g, beta, out):
    tk = tok[...].astype(jnp.bfloat16)
    av = aux[...]
    f = av[:, 0:4].astype(jnp.bfloat16)
    idv = lax.bitcast_convert_type(av[:, 4:7], jnp.int32)
    order = (idv[:, 0:1] == idv[:, 1:2]).astype(jnp.bfloat16)
    iot = lax.broadcasted_iota(jnp.int32, (_ROWS, 4), 1)
    oh = (idv[:, 2:3] == iot).astype(jnp.bfloat16)
    ones = jnp.ones((_ROWS, 1), jnp.bfloat16)
    small = jnp.concatenate([f, order, oh, ones], axis=1)
    x = jnp.concatenate([tk, small], axis=1)
    p = jnp.dot(x, M74[...], preferred_element_type=jnp.float32)
    h = jnp.where(p >= 0, p, 0.01 * p).astype(jnp.bfloat16)
    h = jnp.dot(h, W2[...], preferred_element_type=jnp.float32) + b2[...]
    h = jnp.where(h >= 0, h, 0.01 * h).astype(jnp.bfloat16)
    h = jnp.dot(h, W3[...], preferred_element_type=jnp.float32) + b3[...]
    mu = jnp.mean(h, axis=1, keepdims=True)
    hc = h - mu
    var = jnp.mean(hc * hc, axis=1, keepdims=True)
    out[...] = hc * lax.rsqrt(var + 1e-15) * g[...] + beta[...]


def _mlp_call(tok, aux, M74, W2, b2, W3, b3, g, beta):
    row = lambda i: (i, 0)
    rep = lambda i: (0, 0)
    return pl.pallas_call(
        _mlp_body,
        grid=(_N // _ROWS,),
        in_specs=[
            pl.BlockSpec((_ROWS, _DH), row),
            pl.BlockSpec((_ROWS, 7), row),
            pl.BlockSpec((74, _D // 2), rep),
            pl.BlockSpec((_D // 2, _D), rep),
            pl.BlockSpec((1, _D), rep),
            pl.BlockSpec((_D, _D), rep),
            pl.BlockSpec((1, _D), rep),
            pl.BlockSpec((1, _D), rep),
            pl.BlockSpec((1, _D), rep),
        ],
        out_specs=pl.BlockSpec((_ROWS, _D), row),
        out_shape=jax.ShapeDtypeStruct((_N, _D), jnp.float32),
    )(tok, aux, M74, W2, b2, W3, b3, g, beta)


def kernel(feat, padded_index, etype_ids, token_ids, edge_W, edge_b,
           order_table, etype_table, token_table,
           W1, b1, W2, b2, W3, b3, ln_gamma, ln_beta):
    tok = _token_gather(token_table, token_ids)

    W1a, W1b, W1c, W1d = W1[:64], W1[64:128], W1[128:192], W1[192:]
    Wf = edge_W @ W1a
    dWo = ((order_table[1] - order_table[0]) @ W1b)[None, :]
    We = etype_table @ W1c
    bias1 = (b1 + edge_b @ W1a + order_table[0] @ W1b)[None, :]
    M74 = jnp.concatenate([W1d, Wf, dWo, We, bias1],
                          axis=0).astype(jnp.bfloat16)

    aux = jnp.concatenate(
        [feat.reshape(_N, 4),
         lax.bitcast_convert_type(padded_index.reshape(_N, 2), jnp.float32),
         lax.bitcast_convert_type(etype_ids.reshape(_N, 1), jnp.float32)],
        axis=1)

    out = _mlp_call(tok, aux, M74,
                    W2.astype(jnp.bfloat16), b2[None, :],
                    W3.astype(jnp.bfloat16), b3[None, :],
                    ln_gamma[None, :], ln_beta[None, :])
    return out.reshape(_B, _T, _D)


# speedup vs baseline: 1.1840x; 1.0195x over previous
import functools

import jax
import jax.numpy as jnp
from jax import lax
from jax.experimental import pallas as pl
from jax.experimental.pallas import tpu as pltpu
from jax.experimental.pallas import tpu_sc as plsc

_B, _T, _D, _DH = 16, 4096, 1024, 64
_N = _B * _T
_NC, _NS = 2, 16
_NW = _NC * _NS
_RPW = _N // _NW
_CH = 128
_NCH = _RPW // _CH

_ROWS = 1024


def _gather_body(tab_hbm, ids_hbm, out_hbm, idx_v, rows_v, gsem0, gsem1):
    wid = lax.axis_index("s") * _NC + lax.axis_index("c")
    base = wid * _RPW
    pltpu.sync_copy(ids_hbm.at[wid], idx_v)
    sems = (gsem0, gsem1)
    cps = [None, None]
    cps[0] = pltpu.async_copy(tab_hbm.at[idx_v.at[0]], rows_v.at[0], sems[0])
    for j in range(_NCH):
        if j + 1 < _NCH:
            b = (j + 1) % 2
            cps[b] = pltpu.async_copy(
                tab_hbm.at[idx_v.at[j + 1]], rows_v.at[b], sems[b])
        cps[j % 2].wait()
        pltpu.sync_copy(rows_v.at[j % 2],
                        out_hbm.at[pl.ds(base + j * _CH, _CH)])


def _token_gather(token_table, token_ids):
    ids3 = token_ids.reshape(_NW, _NCH, _CH)
    mesh = plsc.VectorSubcoreMesh(core_axis_name="c", subcore_axis_name="s")
    run = pl.kernel(
        _gather_body,
        out_type=jax.ShapeDtypeStruct((_N, _DH), jnp.float32),
        mesh=mesh,
        scratch_types=[
            pltpu.VMEM((_NCH, _CH), jnp.int32),
            pltpu.VMEM((2, _CH, _DH), jnp.float32),
            pltpu.SemaphoreType.DMA,
            pltpu.SemaphoreType.DMA,
        ],
        compiler_params=pltpu.CompilerParams(use_tc_tiling_on_sc=False),
    )
    return run(token_table, ids3)


_SUB = 512


def _mlp_body(tok, feat, ids, M74, W2, b2, W3, b3, g, beta, out):
    m74 = M74[...]
    w2 = W2[...]
    w3 = W3[...]
    b2v = b2[...].astype(jnp.bfloat16)
    b3v = b3[...]
    gv = g[...]
    bv = beta[...]
    for s in range(_ROWS // _SUB):
        sl = pl.ds(s * _SUB, _SUB)
        tk = tok[sl, :].astype(jnp.bfloat16)
        f = feat[sl, :].astype(jnp.bfloat16)
        idv = ids[sl, :]
        order = (idv[:, 0:1] == idv[:, 1:2]).astype(jnp.bfloat16)
        iot = lax.broadcasted_iota(jnp.int32, (_SUB, 4), 1)
        oh = (idv[:, 2:3] == iot).astype(jnp.bfloat16)
        ones = jnp.ones((_SUB, 1), jnp.bfloat16)
        small = jnp.concatenate([f, order, oh, ones], axis=1)
        x = jnp.concatenate([tk, small], axis=1)
        p = jnp.dot(x, m74,
                    preferred_element_type=jnp.float32).astype(jnp.bfloat16)
        h = jnp.where(p >= 0, p, jnp.bfloat16(0.01) * p)
        h = jnp.dot(h, w2,
                    preferred_element_type=jnp.float32).astype(jnp.bfloat16)
        h = h + b2v
        h = jnp.where(h >= 0, h, jnp.bfloat16(0.01) * h)
        h = jnp.dot(h, w3, preferred_element_type=jnp.float32) + b3v
        mu = jnp.mean(h, axis=1, keepdims=True)
        hc = h - mu
        var = jnp.mean(hc * hc, axis=1, keepdims=True)
        out[sl, :] = hc * lax.rsqrt(var + 1e-15) * gv + bv


def _mlp_call(tok, feat2, ids2, M74, W2, b2, W3, b3, g, beta):
    row = lambda i: (i, 0)
    rep = lambda i: (0, 0)
    return pl.pallas_call(
        _mlp_body,
        grid=(_N // _ROWS,),
        in_specs=[
            pl.BlockSpec((_ROWS, _DH), row),
            pl.BlockSpec((_ROWS, 4), row),
            pl.BlockSpec((_ROWS, 3), row),
            pl.BlockSpec((74, _D // 2), rep),
            pl.BlockSpec((_D // 2, _D), rep),
            pl.BlockSpec((1, _D), rep),
            pl.BlockSpec((_D, _D), rep),
            pl.BlockSpec((1, _D), rep),
            pl.BlockSpec((1, _D), rep),
            pl.BlockSpec((1, _D), rep),
        ],
        out_specs=pl.BlockSpec((_ROWS, _D), row),
        out_shape=jax.ShapeDtypeStruct((_N, _D), jnp.float32),
    )(tok, feat2, ids2, M74, W2, b2, W3, b3, g, beta)


def kernel(feat, padded_index, etype_ids, token_ids, edge_W, edge_b,
           order_table, etype_table, token_table,
           W1, b1, W2, b2, W3, b3, ln_gamma, ln_beta):
    tok = _token_gather(token_table, token_ids)

    W1a, W1b, W1c, W1d = W1[:64], W1[64:128], W1[128:192], W1[192:]
    Wf = edge_W @ W1a
    dWo = ((order_table[1] - order_table[0]) @ W1b)[None, :]
    We = etype_table @ W1c
    bias1 = (b1 + edge_b @ W1a + order_table[0] @ W1b)[None, :]
    M74 = jnp.concatenate([W1d, Wf, dWo, We, bias1],
                          axis=0).astype(jnp.bfloat16)

    feat2 = feat.reshape(_N, 4)
    ids2 = jnp.concatenate(
        [padded_index.reshape(_N, 2), etype_ids.reshape(_N, 1)], axis=1)

    out = _mlp_call(tok, feat2, ids2, M74,
                    W2.astype(jnp.bfloat16), b2[None, :],
                    W3.astype(jnp.bfloat16), b3[None, :],
                    ln_gamma[None, :], ln_beta[None, :])
    return out.reshape(_B, _T, _D)


# speedup vs baseline: 1.2662x; 1.0694x over previous
import functools

import jax
import jax.numpy as jnp
from jax import lax
from jax.experimental import pallas as pl
from jax.experimental.pallas import tpu as pltpu
from jax.experimental.pallas import tpu_sc as plsc

_B, _T, _D, _DH = 16, 4096, 1024, 64
_N = _B * _T
_NC, _NS = 2, 16
_NW = _NC * _NS
_RPW = _N // _NW
_CH = 128
_NCH = _RPW // _CH

_ROWS = 1024


def _gather_body(tab_hbm, ids_hbm, out_hbm, idx_v, rows_v, gsem0, gsem1):
    wid = lax.axis_index("s") * _NC + lax.axis_index("c")
    base = wid * _RPW
    pltpu.sync_copy(ids_hbm.at[wid], idx_v)
    sems = (gsem0, gsem1)
    cps = [None, None]
    cps[0] = pltpu.async_copy(tab_hbm.at[idx_v.at[0]], rows_v.at[0], sems[0])
    for j in range(_NCH):
        if j + 1 < _NCH:
            b = (j + 1) % 2
            cps[b] = pltpu.async_copy(
                tab_hbm.at[idx_v.at[j + 1]], rows_v.at[b], sems[b])
        cps[j % 2].wait()
        pltpu.sync_copy(rows_v.at[j % 2],
                        out_hbm.at[pl.ds(base + j * _CH, _CH)])


def _token_gather(token_table, token_ids):
    tabp = jnp.pad(token_table, ((0, 0), (0, 128 - _DH)))
    ids3 = token_ids.reshape(_NW, _NCH, _CH)
    mesh = plsc.VectorSubcoreMesh(core_axis_name="c", subcore_axis_name="s")
    run = pl.kernel(
        _gather_body,
        out_type=jax.ShapeDtypeStruct((_N, 128), jnp.float32),
        mesh=mesh,
        scratch_types=[
            pltpu.VMEM((_NCH, _CH), jnp.int32),
            pltpu.VMEM((2, _CH, 128), jnp.float32),
            pltpu.SemaphoreType.DMA,
            pltpu.SemaphoreType.DMA,
        ],
        compiler_params=pltpu.CompilerParams(use_tc_tiling_on_sc=False),
    )
    return run(tabp, ids3)


_SUB = 512


def _mlp_body(tok, feat, ids, M74, W2, b2, W3, b3, g, beta, out):
    m74 = M74[...]
    w2 = W2[...]
    w3 = W3[...]
    b2v = b2[...].astype(jnp.bfloat16)
    b3v = b3[...]
    gv = g[...]
    bv = beta[...]
    for s in range(_ROWS // _SUB):
        sl = pl.ds(s * _SUB, _SUB)
        tk = tok[sl, 0:_DH].astype(jnp.bfloat16)
        f = feat[sl, :].astype(jnp.bfloat16)
        idv = ids[sl, :]
        order = (idv[:, 0:1] == idv[:, 1:2]).astype(jnp.bfloat16)
        iot = lax.broadcasted_iota(jnp.int32, (_SUB, 4), 1)
        oh = (idv[:, 2:3] == iot).astype(jnp.bfloat16)
        ones = jnp.ones((_SUB, 1), jnp.bfloat16)
        small = jnp.concatenate([f, order, oh, ones], axis=1)
        x = jnp.concatenate([tk, small], axis=1)
        p = jnp.dot(x, m74,
                    preferred_element_type=jnp.float32).astype(jnp.bfloat16)
        h = jnp.where(p >= 0, p, jnp.bfloat16(0.01) * p)
        h = jnp.dot(h, w2,
                    preferred_element_type=jnp.float32).astype(jnp.bfloat16)
        h = h + b2v
        h = jnp.where(h >= 0, h, jnp.bfloat16(0.01) * h)
        h = jnp.dot(h, w3, preferred_element_type=jnp.float32) + b3v
        mu = jnp.mean(h, axis=1, keepdims=True)
        hc = h - mu
        var = jnp.mean(hc * hc, axis=1, keepdims=True)
        out[sl, :] = hc * lax.rsqrt(var + 1e-15) * gv + bv


def _mlp_call(tok, feat2, ids2, M74, W2, b2, W3, b3, g, beta):
    row = lambda i: (i, 0)
    rep = lambda i: (0, 0)
    return pl.pallas_call(
        _mlp_body,
        grid=(_N // _ROWS,),
        in_specs=[
            pl.BlockSpec((_ROWS, 128), row),
            pl.BlockSpec((_ROWS, 4), row),
            pl.BlockSpec((_ROWS, 3), row),
            pl.BlockSpec((74, _D // 2), rep),
            pl.BlockSpec((_D // 2, _D), rep),
            pl.BlockSpec((1, _D), rep),
            pl.BlockSpec((_D, _D), rep),
            pl.BlockSpec((1, _D), rep),
            pl.BlockSpec((1, _D), rep),
            pl.BlockSpec((1, _D), rep),
        ],
        out_specs=pl.BlockSpec((_ROWS, _D), row),
        out_shape=jax.ShapeDtypeStruct((_N, _D), jnp.float32),
    )(tok, feat2, ids2, M74, W2, b2, W3, b3, g, beta)


def kernel(feat, padded_index, etype_ids, token_ids, edge_W, edge_b,
           order_table, etype_table, token_table,
           W1, b1, W2, b2, W3, b3, ln_gamma, ln_beta):
    tok = _token_gather(token_table, token_ids)

    W1a, W1b, W1c, W1d = W1[:64], W1[64:128], W1[128:192], W1[192:]
    Wf = edge_W @ W1a
    dWo = ((order_table[1] - order_table[0]) @ W1b)[None, :]
    We = etype_table @ W1c
    bias1 = (b1 + edge_b @ W1a + order_table[0] @ W1b)[None, :]
    M74 = jnp.concatenate([W1d, Wf, dWo, We, bias1],
                          axis=0).astype(jnp.bfloat16)

    feat2 = feat.reshape(_N, 4)
    ids2 = jnp.concatenate(
        [padded_index.reshape(_N, 2), etype_ids.reshape(_N, 1)], axis=1)

    out = _mlp_call(tok, feat2, ids2, M74,
                    W2.astype(jnp.bfloat16), b2[None, :],
                    W3.astype(jnp.bfloat16), b3[None, :],
                    ln_gamma[None, :], ln_beta[None, :])
    return out.reshape(_B, _T, _D)


# speedup vs baseline: 1.4149x; 1.1174x over previous
import functools

import jax
import jax.numpy as jnp
from jax import lax
from jax.experimental import pallas as pl
from jax.experimental.pallas import tpu as pltpu
from jax.experimental.pallas import tpu_sc as plsc

_B, _T, _D, _DH = 16, 4096, 1024, 64
_N = _B * _T
_NC, _NS = 2, 16
_NW = _NC * _NS
_RPW = _N // _NW
_CH = 128
_NCH = _RPW // _CH

_ROWS = 1024


def _gather_body(tab_hbm, ids_hbm, out_hbm, idx_v, rows_v, gsem0, gsem1):
    wid = lax.axis_index("s") * _NC + lax.axis_index("c")
    base = wid * _RPW
    pltpu.sync_copy(ids_hbm.at[wid], idx_v)
    sems = (gsem0, gsem1)
    cps = [None, None]
    cps[0] = pltpu.async_copy(tab_hbm.at[idx_v.at[0]], rows_v.at[0], sems[0])
    for j in range(_NCH):
        if j + 1 < _NCH:
            b = (j + 1) % 2
            cps[b] = pltpu.async_copy(
                tab_hbm.at[idx_v.at[j + 1]], rows_v.at[b], sems[b])
        cps[j % 2].wait()
        pltpu.sync_copy(rows_v.at[j % 2],
                        out_hbm.at[pl.ds(base + j * _CH, _CH)])


_TCOL = 3200


def _transpose_body(tt, out):
    out[:, 0:_DH] = tt[...].T


def _widen_table(token_table):
    return pl.pallas_call(
        _transpose_body,
        grid=(TOKEN_ROWS // _TCOL,),
        in_specs=[pl.BlockSpec((_DH, _TCOL), lambda i: (0, i))],
        out_specs=pl.BlockSpec((_TCOL, 128), lambda i: (i, 0)),
        out_shape=jax.ShapeDtypeStruct((TOKEN_ROWS, 128), jnp.float32),
    )(token_table.T)


TOKEN_ROWS = 400000


def _token_gather(token_table, token_ids):
    tabp = _widen_table(token_table)
    ids3 = token_ids.reshape(_NW, _NCH, _CH)
    mesh = plsc.VectorSubcoreMesh(core_axis_name="c", subcore_axis_name="s")
    run = pl.kernel(
        _gather_body,
        out_type=jax.ShapeDtypeStruct((_N, 128), jnp.float32),
        mesh=mesh,
        scratch_types=[
            pltpu.VMEM((_NCH, _CH), jnp.int32),
            pltpu.VMEM((2, _CH, 128), jnp.float32),
            pltpu.SemaphoreType.DMA,
            pltpu.SemaphoreType.DMA,
        ],
        compiler_params=pltpu.CompilerParams(use_tc_tiling_on_sc=False),
    )
    return run(tabp, ids3)


_SUB = 512


def _mlp_body(tok, feat, ids, M74, W2, b2, W3, b3, g, beta, out):
    m74 = M74[...]
    w2 = W2[...]
    w3 = W3[...]
    b2v = b2[...].astype(jnp.bfloat16)
    b3v = b3[...]
    gv = g[...]
    bv = beta[...]
    for s in range(_ROWS // _SUB):
        sl = pl.ds(s * _SUB, _SUB)
        tk = tok[sl, 0:_DH].astype(jnp.bfloat16)
        f = feat[sl, :].astype(jnp.bfloat16)
        idv = ids[sl, :]
        order = (idv[:, 0:1] == idv[:, 1:2]).astype(jnp.bfloat16)
        iot = lax.broadcasted_iota(jnp.int32, (_SUB, 4), 1)
        oh = (idv[:, 2:3] == iot).astype(jnp.bfloat16)
        ones = jnp.ones((_SUB, 1), jnp.bfloat16)
        small = jnp.concatenate([f, order, oh, ones], axis=1)
        x = jnp.concatenate([tk, small], axis=1)
        p = jnp.dot(x, m74,
                    preferred_element_type=jnp.float32).astype(jnp.bfloat16)
        h = jnp.where(p >= 0, p, jnp.bfloat16(0.01) * p)
        h = jnp.dot(h, w2,
                    preferred_element_type=jnp.float32).astype(jnp.bfloat16)
        h = h + b2v
        h = jnp.where(h >= 0, h, jnp.bfloat16(0.01) * h)
        h = jnp.dot(h, w3, preferred_element_type=jnp.float32) + b3v
        mu = jnp.mean(h, axis=1, keepdims=True)
        hc = h - mu
        var = jnp.mean(hc * hc, axis=1, keepdims=True)
        out[sl, :] = hc * lax.rsqrt(var + 1e-15) * gv + bv


def _mlp_call(tok, feat2, ids2, M74, W2, b2, W3, b3, g, beta):
    row = lambda i: (i, 0)
    rep = lambda i: (0, 0)
    return pl.pallas_call(
        _mlp_body,
        grid=(_N // _ROWS,),
        in_specs=[
            pl.BlockSpec((_ROWS, 128), row),
            pl.BlockSpec((_ROWS, 4), row),
            pl.BlockSpec((_ROWS, 3), row),
            pl.BlockSpec((74, _D // 2), rep),
            pl.BlockSpec((_D // 2, _D), rep),
            pl.BlockSpec((1, _D), rep),
            pl.BlockSpec((_D, _D), rep),
            pl.BlockSpec((1, _D), rep),
            pl.BlockSpec((1, _D), rep),
            pl.BlockSpec((1, _D), rep),
        ],
        out_specs=pl.BlockSpec((_ROWS, _D), row),
        out_shape=jax.ShapeDtypeStruct((_N, _D), jnp.float32),
    )(tok, feat2, ids2, M74, W2, b2, W3, b3, g, beta)


def kernel(feat, padded_index, etype_ids, token_ids, edge_W, edge_b,
           order_table, etype_table, token_table,
           W1, b1, W2, b2, W3, b3, ln_gamma, ln_beta):
    tok = _token_gather(token_table, token_ids)

    W1a, W1b, W1c, W1d = W1[:64], W1[64:128], W1[128:192], W1[192:]
    Wf = edge_W @ W1a
    dWo = ((order_table[1] - order_table[0]) @ W1b)[None, :]
    We = etype_table @ W1c
    bias1 = (b1 + edge_b @ W1a + order_table[0] @ W1b)[None, :]
    M74 = jnp.concatenate([W1d, Wf, dWo, We, bias1],
                          axis=0).astype(jnp.bfloat16)

    feat2 = feat.reshape(_N, 4)
    ids2 = jnp.concatenate(
        [padded_index.reshape(_N, 2), etype_ids.reshape(_N, 1)], axis=1)

    out = _mlp_call(tok, feat2, ids2, M74,
                    W2.astype(jnp.bfloat16), b2[None, :],
                    W3.astype(jnp.bfloat16), b3[None, :],
                    ln_gamma[None, :], ln_beta[None, :])
    return out.reshape(_B, _T, _D)


# speedup vs baseline: 1.5724x; 1.1114x over previous
import functools

import jax
import jax.numpy as jnp
from jax import lax
from jax.experimental import pallas as pl
from jax.experimental.pallas import tpu as pltpu
from jax.experimental.pallas import tpu_sc as plsc

_B, _T, _D, _DH = 16, 4096, 1024, 64
_N = _B * _T
_NC, _NS = 2, 16
_NW = _NC * _NS
_RPW = _N // _NW
_CH = 128
_NCH = _RPW // _CH

_ROWS = 1024


def _gather_body(tab_hbm, ids_hbm, out_hbm, idx_v, rows_v, gsem0, gsem1):
    wid = lax.axis_index("s") * _NC + lax.axis_index("c")
    base = wid * _RPW
    pltpu.sync_copy(ids_hbm.at[wid], idx_v)
    sems = (gsem0, gsem1)
    cps = [None, None]
    cps[0] = pltpu.async_copy(tab_hbm.at[idx_v.at[0]], rows_v.at[0], sems[0])
    for j in range(_NCH):
        if j + 1 < _NCH:
            b = (j + 1) % 2
            cps[b] = pltpu.async_copy(
                tab_hbm.at[idx_v.at[j + 1]], rows_v.at[b], sems[b])
        cps[j % 2].wait()
        pltpu.sync_copy(rows_v.at[j % 2],
                        out_hbm.at[pl.ds(base + j * _CH, _CH)])


_TCOL = 16000


def _transpose_body(tt, out):
    out[:, 0:_DH] = tt[...].T


def _widen_table(token_table):
    return pl.pallas_call(
        _transpose_body,
        grid=(TOKEN_ROWS // _TCOL,),
        in_specs=[pl.BlockSpec((_DH, _TCOL), lambda i: (0, i))],
        out_specs=pl.BlockSpec((_TCOL, 128), lambda i: (i, 0)),
        out_shape=jax.ShapeDtypeStruct((TOKEN_ROWS, 128), jnp.float32),
    )(token_table.T)


TOKEN_ROWS = 400000


def _token_gather(token_table, token_ids):
    tabp = _widen_table(token_table)
    ids3 = token_ids.reshape(_NW, _NCH, _CH)
    mesh = plsc.VectorSubcoreMesh(core_axis_name="c", subcore_axis_name="s")
    run = pl.kernel(
        _gather_body,
        out_type=jax.ShapeDtypeStruct((_N, 128), jnp.float32),
        mesh=mesh,
        scratch_types=[
            pltpu.VMEM((_NCH, _CH), jnp.int32),
            pltpu.VMEM((2, _CH, 128), jnp.float32),
            pltpu.SemaphoreType.DMA,
            pltpu.SemaphoreType.DMA,
        ],
        compiler_params=pltpu.CompilerParams(use_tc_tiling_on_sc=False),
    )
    return run(tabp, ids3)


_SUB = 256


def _mlp_body(tok, feat, ids, M74, W2, b2, W3, b3, g, beta, out):
    m74 = M74[...]
    w2 = W2[...]
    w3 = W3[...]
    b2v = b2[...].astype(jnp.bfloat16)
    b3v = b3[...]
    gv = g[...]
    bv = beta[...]
    for s in range(_ROWS // _SUB):
        sl = pl.ds(s * _SUB, _SUB)
        tk = tok[sl, 0:_DH].astype(jnp.bfloat16)
        f = feat[sl, :].astype(jnp.bfloat16)
        idv = ids[sl, :]
        order = (idv[:, 0:1] == idv[:, 1:2]).astype(jnp.bfloat16)
        iot = lax.broadcasted_iota(jnp.int32, (_SUB, 4), 1)
        oh = (idv[:, 2:3] == iot).astype(jnp.bfloat16)
        ones = jnp.ones((_SUB, 1), jnp.bfloat16)
        small = jnp.concatenate([f, order, oh, ones], axis=1)
        x = jnp.concatenate([tk, small], axis=1)
        p = jnp.dot(x, m74,
                    preferred_element_type=jnp.float32).astype(jnp.bfloat16)
        h = jnp.where(p >= 0, p, jnp.bfloat16(0.01) * p)
        h = jnp.dot(h, w2,
                    preferred_element_type=jnp.float32).astype(jnp.bfloat16)
        h = h + b2v
        h = jnp.where(h >= 0, h, jnp.bfloat16(0.01) * h)
        h = jnp.dot(h, w3, preferred_element_type=jnp.float32) + b3v
        mu = jnp.mean(h, axis=1, keepdims=True)
        hc = h - mu
        var = jnp.mean(hc * hc, axis=1, keepdims=True)
        out[sl, :] = hc * lax.rsqrt(var + 1e-15) * gv + bv


def _mlp_call(tok, feat2, ids2, M74, W2, b2, W3, b3, g, beta):
    row = lambda i: (i, 0)
    rep = lambda i: (0, 0)
    return pl.pallas_call(
        _mlp_body,
        grid=(_N // _ROWS,),
        in_specs=[
            pl.BlockSpec((_ROWS, 128), row),
            pl.BlockSpec((_ROWS, 4), row),
            pl.BlockSpec((_ROWS, 3), row),
            pl.BlockSpec((74, _D // 2), rep),
            pl.BlockSpec((_D // 2, _D), rep),
            pl.BlockSpec((1, _D), rep),
            pl.BlockSpec((_D, _D), rep),
            pl.BlockSpec((1, _D), rep),
            pl.BlockSpec((1, _D), rep),
            pl.BlockSpec((1, _D), rep),
        ],
        out_specs=pl.BlockSpec((_ROWS, _D), row),
        out_shape=jax.ShapeDtypeStruct((_N, _D), jnp.float32),
    )(tok, feat2, ids2, M74, W2, b2, W3, b3, g, beta)


def kernel(feat, padded_index, etype_ids, token_ids, edge_W, edge_b,
           order_table, etype_table, token_table,
           W1, b1, W2, b2, W3, b3, ln_gamma, ln_beta):
    tok = _token_gather(token_table, token_ids)

    W1a, W1b, W1c, W1d = W1[:64], W1[64:128], W1[128:192], W1[192:]
    Wf = edge_W @ W1a
    dWo = ((order_table[1] - order_table[0]) @ W1b)[None, :]
    We = etype_table @ W1c
    bias1 = (b1 + edge_b @ W1a + order_table[0] @ W1b)[None, :]
    M74 = jnp.concatenate([W1d, Wf, dWo, We, bias1],
                          axis=0).astype(jnp.bfloat16)

    feat2 = feat.reshape(_N, 4)
    ids2 = jnp.concatenate(
        [padded_index.reshape(_N, 2), etype_ids.reshape(_N, 1)], axis=1)

    out = _mlp_call(tok, feat2, ids2, M74,
                    W2.astype(jnp.bfloat16), b2[None, :],
                    W3.astype(jnp.bfloat16), b3[None, :],
                    ln_gamma[None, :], ln_beta[None, :])
    return out.reshape(_B, _T, _D)


# speedup vs baseline: 1.6107x; 1.0243x over previous
import functools

import jax
import jax.numpy as jnp
from jax import lax
from jax.experimental import pallas as pl
from jax.experimental.pallas import tpu as pltpu
from jax.experimental.pallas import tpu_sc as plsc

_B, _T, _D, _DH = 16, 4096, 1024, 64
_N = _B * _T
_NC, _NS = 2, 16
_NW = _NC * _NS
_RPW = _N // _NW
_CH = 128
_NCH = _RPW // _CH

_ROWS = 2048


def _gather_body(tab_hbm, ids_hbm, out_hbm, idx_v, rows_v, gsem0, gsem1):
    wid = lax.axis_index("s") * _NC + lax.axis_index("c")
    base = wid * _RPW
    pltpu.sync_copy(ids_hbm.at[wid], idx_v)
    sems = (gsem0, gsem1)
    cps = [None, None]
    cps[0] = pltpu.async_copy(tab_hbm.at[idx_v.at[0]], rows_v.at[0], sems[0])
    for j in range(_NCH):
        if j + 1 < _NCH:
            b = (j + 1) % 2
            cps[b] = pltpu.async_copy(
                tab_hbm.at[idx_v.at[j + 1]], rows_v.at[b], sems[b])
        cps[j % 2].wait()
        pltpu.sync_copy(rows_v.at[j % 2],
                        out_hbm.at[pl.ds(base + j * _CH, _CH)])


_TCOL = 16000


def _transpose_body(tt, out):
    out[:, 0:_DH] = tt[...].T


def _widen_table(token_table):
    return pl.pallas_call(
        _transpose_body,
        grid=(TOKEN_ROWS // _TCOL,),
        in_specs=[pl.BlockSpec((_DH, _TCOL), lambda i: (0, i))],
        out_specs=pl.BlockSpec((_TCOL, 128), lambda i: (i, 0)),
        out_shape=jax.ShapeDtypeStruct((TOKEN_ROWS, 128), jnp.float32),
    )(token_table.T)


TOKEN_ROWS = 400000


def _token_gather(token_table, token_ids):
    tabp = _widen_table(token_table)
    ids3 = token_ids.reshape(_NW, _NCH, _CH)
    mesh = plsc.VectorSubcoreMesh(core_axis_name="c", subcore_axis_name="s")
    run = pl.kernel(
        _gather_body,
        out_type=jax.ShapeDtypeStruct((_N, 128), jnp.float32),
        mesh=mesh,
        scratch_types=[
            pltpu.VMEM((_NCH, _CH), jnp.int32),
            pltpu.VMEM((2, _CH, 128), jnp.float32),
            pltpu.SemaphoreType.DMA,
            pltpu.SemaphoreType.DMA,
        ],
        compiler_params=pltpu.CompilerParams(use_tc_tiling_on_sc=False),
    )
    return run(tabp, ids3)


_SUB = 256


def _mlp_body(tok, feat, ids, M74, W2, b2, W3, b3, g, beta, out):
    m74 = M74[...]
    w2 = W2[...]
    w3 = W3[...]
    b2v = b2[...].astype(jnp.bfloat16)
    b3v = b3[...]
    gv = g[...]
    bv = beta[...]
    for s in range(_ROWS // _SUB):
        sl = pl.ds(s * _SUB, _SUB)
        tk = tok[sl, 0:_DH].astype(jnp.bfloat16)
        f = feat[sl, :].astype(jnp.bfloat16)
        idv = ids[sl, :]
        order = (idv[:, 0:1] == idv[:, 1:2]).astype(jnp.bfloat16)
        iot = lax.broadcasted_iota(jnp.int32, (_SUB, 4), 1)
        oh = (idv[:, 2:3] == iot).astype(jnp.bfloat16)
        ones = jnp.ones((_SUB, 1), jnp.bfloat16)
        small = jnp.concatenate([f, order, oh, ones], axis=1)
        x = jnp.concatenate([tk, small], axis=1)
        p = jnp.dot(x, m74,
                    preferred_element_type=jnp.float32).astype(jnp.bfloat16)
        h = jnp.where(p >= 0, p, jnp.bfloat16(0.01) * p)
        h = jnp.dot(h, w2,
                    preferred_element_type=jnp.float32).astype(jnp.bfloat16)
        h = h + b2v
        h = jnp.where(h >= 0, h, jnp.bfloat16(0.01) * h)
        h = jnp.dot(h, w3, preferred_element_type=jnp.float32) + b3v
        mu = jnp.mean(h, axis=1, keepdims=True)
        hc = h - mu
        var = jnp.mean(hc * hc, axis=1, keepdims=True)
        out[sl, :] = hc * lax.rsqrt(var + 1e-15) * gv + bv


def _mlp_call(tok, feat2, ids2, M74, W2, b2, W3, b3, g, beta):
    row = lambda i: (i, 0)
    rep = lambda i: (0, 0)
    return pl.pallas_call(
        _mlp_body,
        grid=(_N // _ROWS,),
        in_specs=[
            pl.BlockSpec((_ROWS, 128), row),
            pl.BlockSpec((_ROWS, 4), row),
            pl.BlockSpec((_ROWS, 3), row),
            pl.BlockSpec((74, _D // 2), rep),
            pl.BlockSpec((_D // 2, _D), rep),
            pl.BlockSpec((1, _D), rep),
            pl.BlockSpec((_D, _D), rep),
            pl.BlockSpec((1, _D), rep),
            pl.BlockSpec((1, _D), rep),
            pl.BlockSpec((1, _D), rep),
        ],
        out_specs=pl.BlockSpec((_ROWS, _D), row),
        out_shape=jax.ShapeDtypeStruct((_N, _D), jnp.float32),
    )(tok, feat2, ids2, M74, W2, b2, W3, b3, g, beta)


def kernel(feat, padded_index, etype_ids, token_ids, edge_W, edge_b,
           order_table, etype_table, token_table,
           W1, b1, W2, b2, W3, b3, ln_gamma, ln_beta):
    tok = _token_gather(token_table, token_ids)

    W1a, W1b, W1c, W1d = W1[:64], W1[64:128], W1[128:192], W1[192:]
    Wf = edge_W @ W1a
    dWo = ((order_table[1] - order_table[0]) @ W1b)[None, :]
    We = etype_table @ W1c
    bias1 = (b1 + edge_b @ W1a + order_table[0] @ W1b)[None, :]
    M74 = jnp.concatenate([W1d, Wf, dWo, We, bias1],
                          axis=0).astype(jnp.bfloat16)

    feat2 = feat.reshape(_N, 4)
    ids2 = jnp.concatenate(
        [padded_index.reshape(_N, 2), etype_ids.reshape(_N, 1)], axis=1)

    out = _mlp_call(tok, feat2, ids2, M74,
                    W2.astype(jnp.bfloat16), b2[None, :],
                    W3.astype(jnp.bfloat16), b3[None, :],
                    ln_gamma[None, :], ln_beta[None, :])
    return out.reshape(_B, _T, _D)
